# baseline (speedup 1.0000x reference)
import jax
import jax.numpy as jnp
from jax import lax
from jax.experimental import pallas as pl
from jax.experimental.pallas import tpu as pltpu

_E = 64
_T = 32768
_CH = 4096
_NCH = _T // _CH
_LANES = 128


def _body(x_hbm, nt_ref, o_ref, b0, b1, sem0, sem1):
    bufs = (b0, b1)
    sems = (sem0, sem1)
    ones = jnp.ones((1, _E), jnp.float32)

    cps = [None] * _NCH
    cps[0] = pltpu.make_async_copy(
        x_hbm.at[:, pl.ds(0, _CH)], bufs[0], sems[0])
    cps[0].start()

    acc = jnp.zeros((_E, _LANES), jnp.float32)
    for c in range(_NCH):
        if c + 1 < _NCH:
            cps[c + 1] = pltpu.make_async_copy(
                x_hbm.at[:, pl.ds((c + 1) * _CH, _CH)],
                bufs[(c + 1) % 2], sems[(c + 1) % 2])
            cps[c + 1].start()
        cps[c].wait()
        buf = bufs[c % 2]
        for j in range(_CH // _LANES):
            ej = jnp.exp(buf[:, j * _LANES:(j + 1) * _LANES])
            dj = lax.dot_general(ones, ej, (((1,), (0,)), ((), ())),
                                 preferred_element_type=jnp.float32)
            acc = acc + ej * (1.0 / dj)

    spe = jnp.sum(acc, axis=1, keepdims=True)
    ntf = nt_ref[...].astype(jnp.float32)
    nts = lax.dot_general(ntf, spe, (((1,), (0,)), ((), ())))[0, 0]
    sum_nt = jnp.sum(ntf)
    balance = (_E / _T) * nts / sum_nt
    sum_s = jnp.sum(spe)
    sum_s2 = jnp.sum(spe * spe)
    m = sum_s / _E
    var = (sum_s2 - _E * m * m) / (_E - 1)
    o_ref[...] = (balance + var / (m * m)).reshape(1, 1)


def kernel(router_logits, num_tokens):
    out = pl.pallas_call(
        _body,
        in_specs=[
            pl.BlockSpec(memory_space=pltpu.MemorySpace.HBM),
            pl.BlockSpec((1, _E), lambda: (0, 0)),
        ],
        out_specs=pl.BlockSpec((1, 1), lambda: (0, 0)),
        out_shape=jax.ShapeDtypeStruct((1, 1), jnp.float32),
        scratch_shapes=[
            pltpu.VMEM((_E, _CH), jnp.float32),
            pltpu.VMEM((_E, _CH), jnp.float32),
            pltpu.SemaphoreType.DMA,
            pltpu.SemaphoreType.DMA,
        ],
    )(router_logits.T, num_tokens.reshape(1, _E))
    return out[0, 0]


# speedup vs baseline: 1.2413x; 1.2413x over previous
import jax
import jax.numpy as jnp
from jax import lax
from jax.experimental import pallas as pl
from jax.experimental.pallas import tpu as pltpu

_E = 64
_T = 32768
_CH = 8192
_NCH = _T // _CH
_LANES = 128


def _body(x_hbm, nt_ref, o_ref, b0, b1, sem0, sem1):
    bufs = (b0, b1)
    sems = (sem0, sem1)
    ones = jnp.ones((1, _E), jnp.float32)

    cps = [None] * _NCH
    cps[0] = pltpu.make_async_copy(
        x_hbm.at[:, pl.ds(0, _CH)], bufs[0], sems[0])
    cps[0].start()

    acc = jnp.zeros((_E, _LANES), jnp.float32)
    for c in range(_NCH):
        if c + 1 < _NCH:
            cps[c + 1] = pltpu.make_async_copy(
                x_hbm.at[:, pl.ds((c + 1) * _CH, _CH)],
                bufs[(c + 1) % 2], sems[(c + 1) % 2])
            cps[c + 1].start()
        cps[c].wait()
        buf = bufs[c % 2]
        for j in range(_CH // _LANES):
            ej = jnp.exp(buf[:, j * _LANES:(j + 1) * _LANES])
            dj = lax.dot_general(ones, ej, (((1,), (0,)), ((), ())),
                                 preferred_element_type=jnp.float32)
            acc = acc + ej * (1.0 / dj)

    spe = jnp.sum(acc, axis=1, keepdims=True)
    ntf = nt_ref[...].astype(jnp.float32)
    nts = lax.dot_general(ntf, spe, (((1,), (0,)), ((), ())))[0, 0]
    sum_nt = jnp.sum(ntf)
    balance = (_E / _T) * nts / sum_nt
    sum_s = jnp.sum(spe)
    sum_s2 = jnp.sum(spe * spe)
    m = sum_s / _E
    var = (sum_s2 - _E * m * m) / (_E - 1)
    o_ref[...] = (balance + var / (m * m)).reshape(1, 1)


def kernel(router_logits, num_tokens):
    out = pl.pallas_call(
        _body,
        in_specs=[
            pl.BlockSpec(memory_space=pltpu.MemorySpace.HBM),
            pl.BlockSpec((1, _E), lambda: (0, 0)),
        ],
        out_specs=pl.BlockSpec((1, 1), lambda: (0, 0)),
        out_shape=jax.ShapeDtypeStruct((1, 1), jnp.float32),
        scratch_shapes=[
            pltpu.VMEM((_E, _CH), jnp.float32),
            pltpu.VMEM((_E, _CH), jnp.float32),
            pltpu.SemaphoreType.DMA,
            pltpu.SemaphoreType.DMA,
        ],
    )(router_logits.T, num_tokens.reshape(1, _E))
    return out[0, 0]


# speedup vs baseline: 1.2735x; 1.0259x over previous
import jax
import jax.numpy as jnp
from jax import lax
from jax.experimental import pallas as pl
from jax.experimental.pallas import tpu as pltpu

_E = 64
_T = 32768
_RC = 8
_NCH = _E // _RC
_LANES = 128


def _body(x_hbm, nt_ref, o_ref, xbuf, dacc, *sems):
    cps = []
    for c in range(_NCH):
        cp = pltpu.make_async_copy(
            x_hbm.at[pl.ds(c * _RC, _RC), :],
            xbuf.at[pl.ds(c * _RC, _RC), :],
            sems[c])
        cp.start()
        cps.append(cp)

    for c in range(_NCH):
        cps[c].wait()
        ec = jnp.exp(xbuf[pl.ds(c * _RC, _RC), :])
        xbuf[pl.ds(c * _RC, _RC), :] = ec
        if c == 0:
            dacc[...] = ec
        else:
            dacc[...] += ec

    r = 1.0 / jnp.sum(dacc[...], axis=0, keepdims=True)

    acc = jnp.zeros((_E, _LANES), jnp.float32)
    for j in range(_T // _LANES):
        acc = acc + (xbuf[:, j * _LANES:(j + 1) * _LANES]
                     * r[:, j * _LANES:(j + 1) * _LANES])
    spe = jnp.sum(acc, axis=1, keepdims=True)
    ntf = nt_ref[...].astype(jnp.float32)
    nts = lax.dot_general(ntf, spe, (((1,), (0,)), ((), ())))[0, 0]
    sum_nt = jnp.sum(ntf)
    balance = (_E / _T) * nts / sum_nt
    sum_s = jnp.sum(spe)
    sum_s2 = jnp.sum(spe * spe)
    m = sum_s / _E
    var = (sum_s2 - _E * m * m) / (_E - 1)
    o_ref[...] = (balance + var / (m * m)).reshape(1, 1)


def kernel(router_logits, num_tokens):
    out = pl.pallas_call(
        _body,
        in_specs=[
            pl.BlockSpec(memory_space=pltpu.MemorySpace.HBM),
            pl.BlockSpec((1, _E), lambda: (0, 0)),
        ],
        out_specs=pl.BlockSpec((1, 1), lambda: (0, 0)),
        out_shape=jax.ShapeDtypeStruct((1, 1), jnp.float32),
        scratch_shapes=[
            pltpu.VMEM((_E, _T), jnp.float32),
            pltpu.VMEM((_RC, _T), jnp.float32),
        ] + [pltpu.SemaphoreType.DMA] * _NCH,
    )(router_logits.T, num_tokens.reshape(1, _E))
    return out[0, 0]
